# Initial kernel scaffold; baseline (speedup 1.0000x reference)
import jax
import jax.numpy as jnp
from jax.experimental import pallas as pl


def kernel(sentence_embeddings, categorical_data, tables, W, b):
    raise NotImplementedError("write your pallas kernel here")


# speedup vs baseline: 1.1552x; 1.1552x over previous
import jax
import jax.numpy as jnp
from jax import lax
from jax.experimental import pallas as pl

N = 16384
F = 26
V = 100000
D = 64
S = 768
K = F * D

_BN = 512


def _mm_body(g_ref, s_ref, w_ref, b_ref, o_ref):
    acc = lax.dot_general(
        g_ref[...], w_ref[...],
        (((1,), (1,)), ((), ())),
        preferred_element_type=jnp.float32,
    )
    o_ref[:, :S] = s_ref[...]
    o_ref[:, S:] = acc + b_ref[...]


def kernel(sentence_embeddings, categorical_data, tables, W, b):
    emb = jax.vmap(lambda t, i: jnp.take(t, i, axis=0))(tables, categorical_data)
    gathered = jnp.transpose(emb, (1, 0, 2)).reshape(N, K)
    out = pl.pallas_call(
        _mm_body,
        grid=(N // _BN,),
        in_specs=[
            pl.BlockSpec((_BN, K), lambda i: (i, 0)),
            pl.BlockSpec((_BN, S), lambda i: (i, 0)),
            pl.BlockSpec((S, K), lambda i: (0, 0)),
            pl.BlockSpec((1, S), lambda i: (0, 0)),
        ],
        out_specs=pl.BlockSpec((_BN, 2 * S), lambda i: (i, 0)),
        out_shape=jax.ShapeDtypeStruct((N, 2 * S), jnp.float32),
    )(gathered, sentence_embeddings, W, b.reshape(1, S))
    return out
